# baseline (speedup 1.0000x reference)
import functools

import jax
import jax.numpy as jnp
from jax import lax
from jax.experimental import pallas as pl
from jax.experimental.pallas import tpu as pltpu
from jax.experimental.pallas import tpu_sc as plsc

_K = 128
_N = 1 + 2 * _K


def _scs_body(buf_hbm, out_hbm, buf_s, out_s):
    @pl.when(lax.axis_index("c") == 0)
    def _():
        pltpu.sync_copy(buf_hbm, buf_s)
        t = buf_s[0]

        def cond(state):
            i, found = state
            return jnp.logical_and(i < _K, jnp.logical_not(found))

        def step(state):
            i, _ = state
            hit = jnp.logical_and(
                t >= buf_s[2 * i + 1], t <= buf_s[2 * i + 2]
            )
            return jnp.where(hit, i, i + 1), hit

        i, found = lax.while_loop(cond, step, (jnp.int32(0), jnp.bool_(False)))
        out_s[0] = jnp.where(found, i, jnp.int32(-1))
        pltpu.sync_copy(out_s, out_hbm)


@jax.jit
def _time_to_interval(t, intervals):
    packed = jnp.concatenate([
        jnp.reshape(jnp.asarray(t, jnp.float32), (1,)),
        jnp.reshape(jnp.asarray(intervals, jnp.float32), (2 * _K,)),
    ])
    run = functools.partial(
        pl.kernel,
        out_type=jax.ShapeDtypeStruct((1,), jnp.int32),
        mesh=plsc.ScalarSubcoreMesh(axis_name="c", num_cores=1),
        compiler_params=pltpu.CompilerParams(
            needs_layout_passes=False, skip_device_barrier=True
        ),
        scratch_types=[
            pltpu.SMEM((_N,), jnp.float32),
            pltpu.SMEM((1,), jnp.int32),
        ],
    )(_scs_body)
    return run(packed)


def kernel(t, intervals):
    out = _time_to_interval(t, intervals)
    return jnp.reshape(out, ())


# speedup vs baseline: 1.0079x; 1.0079x over previous
import functools

import jax
import jax.numpy as jnp
from jax import lax
from jax.experimental import pallas as pl
from jax.experimental.pallas import tpu as pltpu
from jax.experimental.pallas import tpu_sc as plsc

_K = 128


def _scs_body(t_hbm, iv_hbm, out_hbm, t_s, iv_s, out_s, sem_t, sem_iv):
    @pl.when(lax.axis_index("c") == 0)
    def _():
        cp_t = pltpu.make_async_copy(t_hbm, t_s, sem_t)
        cp_iv = pltpu.make_async_copy(iv_hbm, iv_s, sem_iv)
        cp_t.start()
        cp_iv.start()
        cp_t.wait()
        cp_iv.wait()
        t = t_s[0]

        def cond(state):
            i, found = state
            return jnp.logical_and(i < _K, jnp.logical_not(found))

        def step(state):
            i, _ = state
            hit = jnp.logical_and(t >= iv_s[2 * i], t <= iv_s[2 * i + 1])
            return jnp.where(hit, i, i + 1), hit

        i, found = lax.while_loop(cond, step, (jnp.int32(0), jnp.bool_(False)))
        out_s[0] = jnp.where(found, i, jnp.int32(-1))
        pltpu.sync_copy(out_s, out_hbm)


@jax.jit
def _time_to_interval(tv, flat):
    run = functools.partial(
        pl.kernel,
        out_type=jax.ShapeDtypeStruct((1,), jnp.int32),
        mesh=plsc.ScalarSubcoreMesh(axis_name="c", num_cores=1),
        compiler_params=pltpu.CompilerParams(
            needs_layout_passes=False, skip_device_barrier=True
        ),
        scratch_types=[
            pltpu.SMEM((1,), jnp.float32),
            pltpu.SMEM((2 * _K,), jnp.float32),
            pltpu.SMEM((1,), jnp.int32),
            pltpu.SemaphoreType.DMA,
            pltpu.SemaphoreType.DMA,
        ],
    )(_scs_body)
    return run(tv, flat)


def kernel(t, intervals):
    tv = jnp.reshape(jnp.asarray(t, jnp.float32), (1,))
    flat = jnp.reshape(jnp.asarray(intervals, jnp.float32), (2 * _K,))
    out = _time_to_interval(tv, flat)
    return jnp.reshape(out, ())
